# Initial kernel scaffold; baseline (speedup 1.0000x reference)
import jax
import jax.numpy as jnp
from jax.experimental import pallas as pl


def kernel(x_0, x_1, x_2, x_3, params, adjacency_0_rows, adjacency_0_cols, adjacency_1_rows, adjacency_1_cols, adjacency_2_rows, adjacency_2_cols, coadjacency_3_rows, coadjacency_3_cols, incidence_1_rows, incidence_1_cols, incidence_2_rows, incidence_2_cols, incidence_3_rows, incidence_3_cols):
    raise NotImplementedError("write your pallas kernel here")


# speedup vs baseline: 1.2576x; 1.2576x over previous
import functools

import jax
import jax.numpy as jnp
from jax import lax
from jax.experimental import pallas as pl
from jax.experimental.pallas import tpu as pltpu
from jax.experimental.pallas import tpu_sc as plsc

D = 128
NC, NS, LN = 2, 16, 16
NW = NC * NS
NEG = 0.2
CH = 128
EPAD = NW * CH
CH1 = 2048

N0, N1, N2, N3 = 10000, 20000, 10000, 2000

_MESH = plsc.VectorSubcoreMesh(core_axis_name="c", subcore_axis_name="s",
                               num_cores=NC, num_subcores=NS)
_CP = pltpu.CompilerParams(needs_layout_passes=False)


def _rup(x, m):
    return -(-x // m) * m


def _mm_body(x_ref, w_ref, a_ref, m_ref, s_ref):
    m = jnp.dot(x_ref[...], w_ref[...], preferred_element_type=jnp.float32)
    m_ref[...] = m
    s_ref[...] = jnp.dot(m, a_ref[...], preferred_element_type=jnp.float32)


def _project(x, ws, avecs):
    n = x.shape[0]
    k = len(ws)
    K = D * k
    wcat = jnp.concatenate(ws, axis=1)
    a = jnp.zeros((K, D), jnp.float32)
    for col, (b, v) in enumerate(avecs):
        a = a.at[D * b:D * (b + 1), col].set(v)
    bm = 1000
    m, s = pl.pallas_call(
        _mm_body,
        grid=(n // bm,),
        in_specs=[
            pl.BlockSpec((bm, D), lambda i: (i, 0)),
            pl.BlockSpec((D, K), lambda i: (0, 0)),
            pl.BlockSpec((K, D), lambda i: (0, 0)),
        ],
        out_specs=[
            pl.BlockSpec((bm, K), lambda i: (i, 0)),
            pl.BlockSpec((bm, D), lambda i: (i, 0)),
        ],
        out_shape=[
            jax.ShapeDtypeStruct((n, K), jnp.float32),
            jax.ShapeDtypeStruct((n, D), jnp.float32),
        ],
    )(x, wcat, a)
    ms = [m[:, D * i:D * (i + 1)] for i in range(k)]
    svecs = [s[:, j] for j in range(len(avecs))]
    return ms, svecs


def _logit_body(shapes, *refs):
    nrel = len(shapes)
    ins = refs[:4 * nrel]
    outs = refs[4 * nrel:5 * nrel]
    sa_v, idxa_v, idxb_v, gch_v, ssb_v, cidx_v = refs[5 * nrel:]

    c = lax.axis_index("c")
    t = lax.axis_index("s")
    wid = t * NC + c

    for r in range(nrel):
        ep, n_a, n_b = shapes[r]
        idxa, idxb, sa, sb = ins[4 * r:4 * r + 4]
        eout = outs[r]
        ew = ep // NW
        nfull, tail = ew // CH1, ew % CH1

        pltpu.sync_copy(sa, sa_v.at[pl.ds(0, n_a)])

        def chunk(off, sz):
            pltpu.sync_copy(idxa.at[pl.ds(off, sz)], idxa_v.at[pl.ds(0, sz)])
            pltpu.sync_copy(idxb.at[pl.ds(off, sz)], idxb_v.at[pl.ds(0, sz)])

            def sub(j, _):
                jo = j * CH
                for g in range(8):
                    cidx_v[pl.ds(g * LN, LN)] = idxb_v[pl.ds(jo + g * LN, LN)]
                pltpu.sync_copy(sb.at[cidx_v], ssb_v)
                for g in range(8):
                    aa = plsc.load_gather(
                        sa_v, [idxa_v[pl.ds(jo + g * LN, LN)]])
                    x = aa + ssb_v[pl.ds(g * LN, LN)]
                    gch_v[pl.ds(jo + g * LN, LN)] = jnp.maximum(x, NEG * x)
                return 0
            lax.fori_loop(0, sz // CH, sub, 0)
            pltpu.sync_copy(gch_v.at[pl.ds(0, sz)], eout.at[pl.ds(off, sz)])

        def _pk(k, _):
            chunk(wid * ew + k * CH1, CH1)
            return 0
        lax.fori_loop(0, nfull, _pk, 0)
        if tail:
            chunk(wid * ew + nfull * CH1, tail)


def _logits(rels, dep=None):
    shapes = tuple((r[0].shape[0], r[2].shape[0], r[3].shape[0]) for r in rels)
    max_na = max(s[1] for s in shapes)
    if dep is not None:
        i0, _ = lax.optimization_barrier((rels[0][0], dep))
        rels = [(i0,) + tuple(rels[0][1:])] + list(rels[1:])
    flat = []
    for r in rels:
        flat.extend(r)
    body = functools.partial(_logit_body, shapes)
    outs = pl.kernel(
        body,
        out_type=[jax.ShapeDtypeStruct((s[0],), jnp.float32) for s in shapes],
        mesh=_MESH,
        compiler_params=_CP,
        scratch_types=[
            pltpu.VMEM((max_na,), jnp.float32),
            pltpu.VMEM((CH1,), jnp.int32),
            pltpu.VMEM((CH1,), jnp.int32),
            pltpu.VMEM((CH1,), jnp.float32),
            pltpu.VMEM((CH,), jnp.float32),
            pltpu.VMEM((CH,), jnp.int32),
        ],
    )(*flat)
    return outs if isinstance(outs, (list, tuple)) else [outs]


def _agg_body(shapes, half, *refs):
    nrel = len(shapes)
    ins = refs[:4 * nrel]
    out_ref = refs[4 * nrel]
    (idxd_v, idxs_v, attc_v, csrc_v, orow_v, gbuf, zbuf, sp_out) = \
        refs[4 * nrel + 1:]
    spr = sp_out.shape[0]

    c = lax.axis_index("c")
    t = lax.axis_index("s")
    base = c * half
    tr = half

    zero16 = jnp.zeros((LN,), jnp.float32)
    for i in range(8):
        for cc in range(8):
            zbuf[i, pl.ds(cc * LN, LN)] = zero16

    zr = spr // NS

    def _z(i, _):
        pltpu.sync_copy(zbuf, sp_out.at[pl.ds(t * zr + i * 8, 8)])
        return 0
    lax.fori_loop(0, zr // 8, _z, 0)
    plsc.subcore_barrier()

    for r in range(nrel):
        ep, n_s = shapes[r]
        dsti, srci, atti, msrc = ins[4 * r:4 * r + 4]
        et = ep // NS
        nfull, tail = et // CH1, et % CH1

        def chunk(off, sz):
            pltpu.sync_copy(dsti.at[pl.ds(off, sz)], idxd_v.at[pl.ds(0, sz)])
            pltpu.sync_copy(srci.at[pl.ds(off, sz)], idxs_v.at[pl.ds(0, sz)])
            pltpu.sync_copy(atti.at[pl.ds(off, sz)], attc_v.at[pl.ds(0, sz)])

            def sub(j, _):
                jo = j * CH
                for g in range(8):
                    rr = idxd_v[pl.ds(jo + g * LN, LN)]
                    ridx = rr - base
                    inr = (rr >= base) & (ridx < half)
                    orow_v[pl.ds(g * LN, LN)] = jnp.where(inr, ridx, tr)
                    csrc_v[pl.ds(g * LN, LN)] = idxs_v[pl.ds(jo + g * LN, LN)]
                pltpu.sync_copy(msrc.at[csrc_v], gbuf)

                def _sc(q, _):
                    for jj in range(LN):
                        row = q * LN + jj
                        av = plsc.load_gather(
                            attc_v, [jnp.full((LN,), jo + row, jnp.int32)])
                        for cc in range(8):
                            gbuf[row, pl.ds(cc * LN, LN)] = (
                                gbuf[row, pl.ds(cc * LN, LN)] * av)
                    return 0
                lax.fori_loop(0, CH // LN, _sc, 0)
                pltpu.sync_copy(gbuf, sp_out.at[orow_v], add=True)
                return 0
            lax.fori_loop(0, sz // CH, sub, 0)

        def _pk(k, _):
            chunk(t * et + k * CH1, CH1)
            return 0
        lax.fori_loop(0, nfull, _pk, 0)
        if tail:
            chunk(t * et + nfull * CH1, tail)
    plsc.subcore_barrier()

    rpt = (half // NS) & ~7
    pltpu.sync_copy(sp_out.at[pl.ds(t * rpt, rpt)],
                    out_ref.at[pl.ds(base + t * rpt, rpt)])
    rem = half - NS * rpt
    if rem:
        @pl.when(t == 0)
        def _():
            pltpu.sync_copy(sp_out.at[pl.ds(NS * rpt, rem)],
                            out_ref.at[pl.ds(base + NS * rpt, rem)])


def _aggregate(n_out, rels, dep=None):
    half = n_out // NC
    spr = _rup(half + 16, 128)
    shapes = tuple((r[0].shape[0], r[3].shape[0]) for r in rels)
    if dep is not None:
        d0, _ = lax.optimization_barrier((rels[0][0], dep))
        rels = [(d0,) + tuple(rels[0][1:])] + list(rels[1:])
    flat = []
    for r in rels:
        flat.extend(r)
    body = functools.partial(_agg_body, shapes, half)
    return pl.kernel(
        body,
        out_type=jax.ShapeDtypeStruct((n_out, D), jnp.float32),
        mesh=_MESH,
        compiler_params=_CP,
        scratch_types=[
            pltpu.VMEM((CH1,), jnp.int32),
            pltpu.VMEM((CH1,), jnp.int32),
            pltpu.VMEM((CH1,), jnp.float32),
            pltpu.VMEM((CH,), jnp.int32),
            pltpu.VMEM((CH,), jnp.int32),
            pltpu.VMEM((CH, D), jnp.float32),
            pltpu.VMEM((8, D), jnp.float32),
            pltpu.VMEM_SHARED((spr, D), jnp.float32),
        ],
    )(*flat)


def _pad(x, ep, val):
    e = x.shape[0]
    if ep == e:
        return x
    return jnp.concatenate([x, jnp.full((ep - e,), val, x.dtype)])


def _att(e_pad, ne, dst, n):
    e = e_pad[:ne]
    den = jax.ops.segment_sum(e, dst, num_segments=n)
    att = e / den[dst]
    return _pad(att, e_pad.shape[0], 0.0)


def kernel(x_0, x_1, x_2, x_3, params,
           adjacency_0_rows, adjacency_0_cols, adjacency_1_rows,
           adjacency_1_cols, adjacency_2_rows, adjacency_2_cols,
           coadjacency_3_rows, coadjacency_3_cols, incidence_1_rows,
           incidence_1_cols, incidence_2_rows, incidence_2_cols,
           incidence_3_rows, incidence_3_cols):
    p = params

    def ah(name):
        a = p[name][:, 0]
        return a[:D], a[D:]

    edges = {
        'a0': (adjacency_0_rows, adjacency_0_cols, N0, N0),
        'a1': (adjacency_1_rows, adjacency_1_cols, N1, N1),
        'a2': (adjacency_2_rows, adjacency_2_cols, N2, N2),
        'c3': (coadjacency_3_rows, coadjacency_3_cols, N3, N3),
        'i1': (incidence_1_rows, incidence_1_cols, N0, N1),
        'i2': (incidence_2_rows, incidence_2_cols, N1, N2),
        'i3': (incidence_3_rows, incidence_3_cols, N2, N3),
    }
    epd, r0, c0, rn, cn = {}, {}, {}, {}, {}
    for k, (rw, cl, nr, ncl) in edges.items():
        ep = _rup(rw.shape[0], EPAD)
        epd[k] = ep
        r0[k] = _pad(rw, ep, 0)
        c0[k] = _pad(cl, ep, 0)
        rn[k] = _pad(rw, ep, nr)
        cn[k] = _pad(cl, ep, ncl)
    ne = {k: edges[k][0].shape[0] for k in edges}

    h0a1, h0a2 = ah('hbs0_l1_a')
    a01s, a01t = ah('hbns01_l1_a')
    a12s, a12t = ah('hbns12_l1_a')
    a23s, a23t = ah('hbns23_l1_a')
    (m00, t01), (s00d, s00s, st01) = _project(
        x_0, [p['hbs0_l1_W'], p['hbns01_l1_wt']],
        [(0, h0a1), (0, h0a2), (1, a01t)])
    (s01, t12), (ss01, st12) = _project(
        x_1, [p['hbns01_l1_ws'], p['hbns12_l1_wt']],
        [(0, a01s), (1, a12t)])
    (s12, t23), (ss12, st23) = _project(
        x_2, [p['hbns12_l1_ws'], p['hbns23_l1_wt']],
        [(0, a12s), (1, a23t)])
    (s23,), (ss23,) = _project(x_3, [p['hbns23_l1_ws']], [(0, a23s)])

    e_a0, e_i1, e_i2, e_i3 = _logits([
        (r0['a0'], c0['a0'], s00d, s00s),
        (r0['i1'], c0['i1'], st01, ss01),
        (r0['i2'], c0['i2'], st12, ss12),
        (r0['i3'], c0['i3'], st23, ss23),
    ])
    att_a0 = _att(e_a0, ne['a0'], adjacency_0_rows, N0)
    att_i1e = _att(e_i1, ne['i1'], incidence_1_rows, N0)
    att_i1f = _att(e_i1, ne['i1'], incidence_1_cols, N1)
    att_i2e = _att(e_i2, ne['i2'], incidence_2_rows, N1)
    att_i2f = _att(e_i2, ne['i2'], incidence_2_cols, N2)
    att_i3e = _att(e_i3, ne['i3'], incidence_3_rows, N2)
    att_i3f = _att(e_i3, ne['i3'], incidence_3_cols, N3)

    x0l1 = _aggregate(N0, [
        (rn['a0'], c0['a0'], att_a0, m00),
        (rn['i1'], c0['i1'], att_i1e, s01),
    ], dep=e_a0)
    x1l1 = _aggregate(N1, [
        (cn['i1'], r0['i1'], att_i1f, t01),
        (rn['i2'], c0['i2'], att_i2e, s12),
    ], dep=x0l1)
    x2l1 = _aggregate(N2, [
        (cn['i2'], r0['i2'], att_i2f, t12),
        (rn['i3'], c0['i3'], att_i3e, s23),
    ], dep=x1l1)
    x3l1 = _aggregate(N3, [
        (cn['i3'], r0['i3'], att_i3f, t23),
    ], dep=x2l1)

    g0a1, g0a2 = ah('hbs0_l2_a')
    g1a1, g1a2 = ah('hbs1_l2_a')
    g2a1, g2a2 = ah('hbs2_l2_a')
    g3a1, g3a2 = ah('hbs3_l2_a')
    b01s, b01t = ah('hbns01_l2_a')
    b12s, b12t = ah('hbns12_l2_a')
    b23s, b23t = ah('hbns23_l2_a')
    (n00, u01), (n00d, n00s, ut01) = _project(
        x0l1, [p['hbs0_l2_W'], p['hbns01_l2_wt']],
        [(0, g0a1), (0, g0a2), (1, b01t)])
    (n11, v01, u12), (n11d, n11s, vs01, ut12) = _project(
        x1l1, [p['hbs1_l2_W'], p['hbns01_l2_ws'], p['hbns12_l2_wt']],
        [(0, g1a1), (0, g1a2), (1, b01s), (2, b12t)])
    (n22, v12, u23), (n22d, n22s, vs12, ut23) = _project(
        x2l1, [p['hbs2_l2_W'], p['hbns12_l2_ws'], p['hbns23_l2_wt']],
        [(0, g2a1), (0, g2a2), (1, b12s), (2, b23t)])
    (n33, v23), (n33d, n33s, vs23) = _project(
        x3l1, [p['hbs3_l2_W'], p['hbns23_l2_ws']],
        [(0, g3a1), (0, g3a2), (1, b23s)])
    del v01, v12, v23

    f_a0, f_a1, f_a2, f_c3, f_i1, f_i2, f_i3 = _logits([
        (r0['a0'], c0['a0'], n00d, n00s),
        (r0['a1'], c0['a1'], n11d, n11s),
        (r0['a2'], c0['a2'], n22d, n22s),
        (r0['c3'], c0['c3'], n33d, n33s),
        (c0['i1'], r0['i1'], vs01, ut01),
        (c0['i2'], r0['i2'], vs12, ut12),
        (c0['i3'], r0['i3'], vs23, ut23),
    ], dep=x3l1)
    att2_a0 = _att(f_a0, ne['a0'], adjacency_0_rows, N0)
    att2_a1 = _att(f_a1, ne['a1'], adjacency_1_rows, N1)
    att2_a2 = _att(f_a2, ne['a2'], adjacency_2_rows, N2)
    att2_c3 = _att(f_c3, ne['c3'], coadjacency_3_rows, N3)
    att2_i1 = _att(f_i1, ne['i1'], incidence_1_cols, N1)
    att2_i2 = _att(f_i2, ne['i2'], incidence_2_cols, N2)
    att2_i3 = _att(f_i3, ne['i3'], incidence_3_cols, N3)

    y0 = _aggregate(N0, [
        (rn['a0'], c0['a0'], att2_a0, n00),
    ], dep=f_a0)
    y1 = _aggregate(N1, [
        (rn['a1'], c0['a1'], att2_a1, n11),
        (cn['i1'], r0['i1'], att2_i1, u01),
    ], dep=y0)
    y2 = _aggregate(N2, [
        (rn['a2'], c0['a2'], att2_a2, n22),
        (cn['i2'], r0['i2'], att2_i2, u12),
    ], dep=y1)
    y3 = _aggregate(N3, [
        (rn['c3'], c0['c3'], att2_c3, n33),
        (cn['i3'], r0['i3'], att2_i3, u23),
    ], dep=y2)
    return (y0, y1, y2, y3)
